# Initial kernel scaffold; baseline (speedup 1.0000x reference)
import jax
import jax.numpy as jnp
from jax.experimental import pallas as pl


def kernel(x, w):
    raise NotImplementedError("write your pallas kernel here")


# speedup vs baseline: 1.0218x; 1.0218x over previous
import functools

import jax
import jax.numpy as jnp
from jax import lax
from jax.experimental import pallas as pl
from jax.experimental.pallas import tpu as pltpu
from jax.experimental.pallas import tpu_sc as plsc

_info = plsc.get_sparse_core_info()
_NC, _NS = _info.num_cores, _info.num_subcores
_NW = _NC * _NS

_CHUNK = 128


def _embed_kernel(n_chunks, d, idx_hbm, table_hbm, out_hbm, idx_v, rows_v, sem):
    wid = lax.axis_index("s") * _NC + lax.axis_index("c")
    pltpu.sync_copy(idx_hbm.at[wid], idx_v)
    base = wid * (n_chunks * _CHUNK)

    def body(g, carry):
        pltpu.async_copy(table_hbm.at[idx_v.at[g]], rows_v, sem).wait()
        pltpu.sync_copy(rows_v, out_hbm.at[pl.ds(base + g * _CHUNK, _CHUNK)])
        return carry

    lax.fori_loop(0, n_chunks, body, 0)


def kernel(x, w):
    b = x.size
    d = w.shape[1]
    assert b % (_NW * _CHUNK) == 0
    n_chunks = b // (_NW * _CHUNK)
    idx = x.reshape(_NW, n_chunks, _CHUNK).astype(jnp.int32)

    mesh = plsc.VectorSubcoreMesh(core_axis_name="c", subcore_axis_name="s")
    k = functools.partial(
        pl.kernel,
        mesh=mesh,
        out_type=jax.ShapeDtypeStruct((b, d), jnp.float32),
        scratch_types=[
            pltpu.VMEM((n_chunks, _CHUNK), jnp.int32),
            pltpu.VMEM((_CHUNK, d), jnp.float32),
            pltpu.SemaphoreType.DMA,
        ],
        compiler_params=pltpu.CompilerParams(use_tc_tiling_on_sc=False),
    )(functools.partial(_embed_kernel, n_chunks, d))
    out = k(idx, w)
    return out.reshape(*x.shape, d)


# speedup vs baseline: 1.1126x; 1.0889x over previous
import functools

import jax
import jax.numpy as jnp
from jax import lax
from jax.experimental import pallas as pl
from jax.experimental.pallas import tpu as pltpu
from jax.experimental.pallas import tpu_sc as plsc

_info = plsc.get_sparse_core_info()
_NC, _NS = _info.num_cores, _info.num_subcores
_NW = _NC * _NS

_CHUNK = 128
_K = 10
_GROWS = _K * _CHUNK


def _embed_kernel(n_chunks, d, idx_hbm, table_hbm, out_hbm,
                  idx_v, rows0, rows1, sem0, sem1):
    wid = lax.axis_index("s") * _NC + lax.axis_index("c")
    pltpu.sync_copy(idx_hbm.at[wid], idx_v)
    base = wid * (n_chunks * _CHUNK)
    n_groups = n_chunks // _K

    def issue(g, buf, sem):
        for j in range(_K):
            pltpu.async_copy(table_hbm.at[idx_v.at[g * _K + j]],
                             buf.at[pl.ds(j * _CHUNK, _CHUNK)], sem)

    def drain(buf, sem):
        for j in range(_K):
            pltpu.make_async_copy(table_hbm.at[idx_v.at[j]],
                                  buf.at[pl.ds(j * _CHUNK, _CHUNK)],
                                  sem).wait()

    issue(0, rows0, sem0)

    def body(i, carry):
        issue(2 * i + 1, rows1, sem1)
        drain(rows0, sem0)
        pltpu.sync_copy(rows0, out_hbm.at[pl.ds(base + (2 * i) * _GROWS,
                                                _GROWS)])

        @pl.when(i < n_groups // 2 - 1)
        def _():
            issue(2 * i + 2, rows0, sem0)

        drain(rows1, sem1)
        pltpu.sync_copy(rows1, out_hbm.at[pl.ds(base + (2 * i + 1) * _GROWS,
                                                _GROWS)])
        return carry

    lax.fori_loop(0, n_groups // 2, body, 0)


def kernel(x, w):
    b = x.size
    d = w.shape[1]
    assert b % (_NW * _CHUNK * _K * 2) == 0
    n_chunks = b // (_NW * _CHUNK)
    idx = x.reshape(_NW, n_chunks, _CHUNK).astype(jnp.int32)

    mesh = plsc.VectorSubcoreMesh(core_axis_name="c", subcore_axis_name="s")
    k = functools.partial(
        pl.kernel,
        mesh=mesh,
        out_type=jax.ShapeDtypeStruct((b, d), jnp.float32),
        scratch_types=[
            pltpu.VMEM((n_chunks, _CHUNK), jnp.int32),
            pltpu.VMEM((_GROWS, d), jnp.float32),
            pltpu.VMEM((_GROWS, d), jnp.float32),
            pltpu.SemaphoreType.DMA,
            pltpu.SemaphoreType.DMA,
        ],
        compiler_params=pltpu.CompilerParams(use_tc_tiling_on_sc=False),
    )(functools.partial(_embed_kernel, n_chunks, d))
    out = k(idx, w)
    return out.reshape(*x.shape, d)


# speedup vs baseline: 1.6466x; 1.4799x over previous
import functools

import jax
import jax.numpy as jnp
import numpy as np
from jax import lax
from jax.experimental import pallas as pl
from jax.experimental.pallas import tpu as pltpu
from jax.experimental.pallas import tpu_sc as plsc

_info = plsc.get_sparse_core_info()
_NC, _NS = _info.num_cores, _info.num_subcores
_NW = _NC * _NS

_LANES = 128
_IOTA16 = np.arange(16, dtype=np.int32)


def _embed_kernel(n_blocks, idx_hbm, table_hbm, o5_hbm,
                  idx_v, g0, g1, ob0, ob1, gsem0, gsem1, wsem0, wsem1):
    wid = lax.axis_index("s") * _NC + lax.axis_index("c")
    pltpu.sync_copy(idx_hbm.at[wid], idx_v)

    def issue(j, buf, sem):
        pltpu.async_copy(table_hbm.at[idx_v.at[j]], buf, sem)

    def drain_gather(buf, sem):
        pltpu.make_async_copy(table_hbm.at[idx_v.at[0]], buf, sem).wait()

    def transpose_block(gbuf, obuf):
        iota = lax.iota(jnp.int32, 16)

        def body(d, carry):
            tr = d // 8
            sl = d % 8
            dv = jnp.full((16,), d, jnp.int32)
            for lb in range(8):
                vals = plsc.load_gather(gbuf, [iota + (16 * lb), dv])
                obuf[tr, sl, pl.ds(16 * lb, 16)] = vals
            return carry

        lax.fori_loop(0, 32, body, 0)

    def write_out(j, obuf, wsem):
        beta = wid * n_blocks + j
        s = beta // 128
        k = beta % 128
        for tr in range(4):
            pltpu.async_copy(obuf.at[tr], o5_hbm.at[s, tr, k], wsem)

    def drain_write(obuf, wsem):
        for tr in range(4):
            pltpu.make_async_copy(obuf.at[tr], o5_hbm.at[0, tr, 0],
                                  wsem).wait()

    issue(0, g0, gsem0)

    def body(i, carry):
        issue(2 * i + 1, g1, gsem1)
        drain_gather(g0, gsem0)

        @pl.when(i > 0)
        def _():
            drain_write(ob0, wsem0)

        transpose_block(g0, ob0)
        write_out(2 * i, ob0, wsem0)

        @pl.when(i < n_blocks // 2 - 1)
        def _():
            issue(2 * i + 2, g0, gsem0)

        drain_gather(g1, gsem1)

        @pl.when(i > 0)
        def _():
            drain_write(ob1, wsem1)

        transpose_block(g1, ob1)
        write_out(2 * i + 1, ob1, wsem1)
        return carry

    lax.fori_loop(0, n_blocks // 2, body, 0)
    drain_write(ob0, wsem0)
    drain_write(ob1, wsem1)


def kernel(x, w):
    b, s_len = x.shape
    d = w.shape[1]
    n_blk_rows = b // _LANES
    total_blocks = s_len * n_blk_rows
    assert total_blocks % (2 * _NW) == 0
    n_blocks = total_blocks // _NW
    idx = x.astype(jnp.int32).T.reshape(_NW, n_blocks, _LANES)

    mesh = plsc.VectorSubcoreMesh(core_axis_name="c", subcore_axis_name="s")
    k = functools.partial(
        pl.kernel,
        mesh=mesh,
        out_type=jax.ShapeDtypeStruct((s_len, d // 8, n_blk_rows, 8, _LANES),
                                      jnp.float32),
        scratch_types=[
            pltpu.VMEM((n_blocks, _LANES), jnp.int32),
            pltpu.VMEM((_LANES, d), jnp.float32),
            pltpu.VMEM((_LANES, d), jnp.float32),
            pltpu.VMEM((d // 8, 8, _LANES), jnp.float32),
            pltpu.VMEM((d // 8, 8, _LANES), jnp.float32),
            pltpu.SemaphoreType.DMA,
            pltpu.SemaphoreType.DMA,
            pltpu.SemaphoreType.DMA,
            pltpu.SemaphoreType.DMA,
        ],
        compiler_params=pltpu.CompilerParams(use_tc_tiling_on_sc=False,
                                             needs_layout_passes=False),
    )(functools.partial(_embed_kernel, n_blocks))
    o5 = k(idx, w)
    return o5.transpose(2, 4, 0, 1, 3).reshape(b, s_len, d)
